# baseline (device time: 11622 ns/iter reference)
import jax
import jax.numpy as jnp
from jax import lax
from jax.experimental import pallas as pl
from jax.experimental.pallas import tpu as pltpu

N_DEV = 4
EPS = 1e-5


def kernel(x, gamma, beta):
    m, n_local = x.shape
    n_global = n_local * N_DEV
    nh = n_local // 2

    def body(
        x_hbm, g_hbm, b_hbm, out_hbm,
        x_vmem, g_vmem, b_vmem, out_vmem,
        comm_ref, load_sems, store_sem, send_sems, recv_sems,
    ):
        my = lax.axis_index("i")
        peers = [lax.rem(my + d, N_DEV) for d in range(1, N_DEV)]

        halves = [pl.ds(0, nh), pl.ds(nh, nh)]
        x_loads = []
        for c, cols in enumerate(halves):
            cp = pltpu.make_async_copy(
                x_hbm.at[:, cols], x_vmem.at[:, cols], load_sems.at[c]
            )
            cp.start()
            x_loads.append(cp)
        g_load = pltpu.make_async_copy(g_hbm, g_vmem, load_sems.at[2])
        b_load = pltpu.make_async_copy(b_hbm, b_vmem, load_sems.at[3])
        g_load.start()
        b_load.start()


        x_loads[0].wait()
        xh0 = x_vmem[:, halves[0]]
        s1 = jnp.sum(xh0, axis=1, keepdims=True)
        s2 = jnp.sum(xh0 * xh0, axis=1, keepdims=True)
        x_loads[1].wait()
        xh1 = x_vmem[:, halves[1]]
        s1 = s1 + jnp.sum(xh1, axis=1, keepdims=True)
        s2 = s2 + jnp.sum(xh1 * xh1, axis=1, keepdims=True)
        comm_ref[my] = jnp.concatenate([s1, s2], axis=1).T

        sends = []
        for d, peer in [(1, peers[1]), (0, peers[0]), (2, peers[2])]:
            rdma = pltpu.make_async_remote_copy(
                src_ref=comm_ref.at[my],
                dst_ref=comm_ref.at[my],
                send_sem=send_sems.at[d],
                recv_sem=recv_sems.at[my],
                device_id=(peer,),
                device_id_type=pl.DeviceIdType.MESH,
            )
            rdma.start()
            sends.append(rdma)

        g_load.wait()
        b_load.wait()
        g = g_vmem[:].reshape(1, -1)
        xf = x_vmem[:, :]
        xg = xf * g

        for d, peer in enumerate(peers):
            recv = pltpu.make_async_remote_copy(
                src_ref=comm_ref.at[peer],
                dst_ref=comm_ref.at[peer],
                send_sem=send_sems.at[d],
                recv_sem=recv_sems.at[peer],
                device_id=(peer,),
                device_id_type=pl.DeviceIdType.MESH,
            )
            recv.wait_recv()

        total = (
            comm_ref[0] + comm_ref[1] + comm_ref[2] + comm_ref[3]
        )
        mean = total[0:1, :].T / n_global
        var = total[1:2, :].T / n_global - mean * mean
        inv = lax.rsqrt(var + EPS)
        b = b_vmem[:].reshape(1, -1)
        out_vmem[:, :] = (xg * inv - (mean * inv) * g + b).astype(jnp.bfloat16)
        out_store = pltpu.make_async_copy(out_vmem, out_hbm, store_sem)
        out_store.start()
        out_store.wait()
        for rdma in sends:
            rdma.wait_send()

    return pl.pallas_call(
        body,
        out_shape=jax.ShapeDtypeStruct((m, n_local), jnp.bfloat16),
        in_specs=[
            pl.BlockSpec(memory_space=pltpu.MemorySpace.HBM),
            pl.BlockSpec(memory_space=pltpu.MemorySpace.HBM),
            pl.BlockSpec(memory_space=pltpu.MemorySpace.HBM),
        ],
        out_specs=pl.BlockSpec(memory_space=pltpu.MemorySpace.HBM),
        scratch_shapes=[
            pltpu.VMEM((m, n_local), jnp.float32),
            pltpu.VMEM((n_local,), jnp.float32),
            pltpu.VMEM((n_local,), jnp.float32),
            pltpu.VMEM((m, n_local), jnp.bfloat16),
            pltpu.VMEM((N_DEV, 2, m), jnp.float32),
            pltpu.SemaphoreType.DMA((4,)),
            pltpu.SemaphoreType.DMA,
            pltpu.SemaphoreType.DMA((N_DEV - 1,)),
            pltpu.SemaphoreType.DMA((N_DEV,)),
        ],
    )(
        pltpu.with_memory_space_constraint(x, pltpu.MemorySpace.HBM),
        pltpu.with_memory_space_constraint(gamma, pltpu.MemorySpace.HBM),
        pltpu.with_memory_space_constraint(beta, pltpu.MemorySpace.HBM),
    )


# device time: 8195 ns/iter; 1.4182x vs baseline; 1.4182x over previous
import jax
import jax.numpy as jnp
from jax import lax
from jax.experimental import pallas as pl
from jax.experimental.pallas import tpu as pltpu

N_DEV = 4
EPS = 1e-5


def kernel(x, gamma, beta):
    m, n_local = x.shape
    n_global = n_local * N_DEV
    nh = n_local // 2

    def body(
        x_hbm, g_hbm, b_hbm, out_hbm,
        x_vmem, g_vmem, b_vmem, out_vmem,
        comm_ref, load_sems, store_sem, send_sems, recv_sems,
    ):
        my = lax.axis_index("i")
        peers = [lax.rem(my + d, N_DEV) for d in range(1, N_DEV)]

        halves = [pl.ds(0, nh), pl.ds(nh, nh)]
        x_loads = []
        for c, cols in enumerate(halves):
            cp = pltpu.make_async_copy(
                x_hbm.at[:, cols], x_vmem.at[:, cols], load_sems.at[c]
            )
            cp.start()
            x_loads.append(cp)
        g_load = pltpu.make_async_copy(g_hbm, g_vmem, load_sems.at[2])
        b_load = pltpu.make_async_copy(b_hbm, b_vmem, load_sems.at[3])
        g_load.start()
        b_load.start()

        barrier_sem = pltpu.get_barrier_semaphore()
        for peer in peers:
            pl.semaphore_signal(
                barrier_sem, inc=1,
                device_id=(peer,), device_id_type=pl.DeviceIdType.MESH,
            )
        pl.semaphore_wait(barrier_sem, N_DEV - 1)

        x_loads[0].wait()
        xh0 = x_vmem[:, halves[0]]
        s1 = jnp.sum(xh0, axis=1, keepdims=True)
        s2 = jnp.sum(xh0 * xh0, axis=1, keepdims=True)
        x_loads[1].wait()
        xh1 = x_vmem[:, halves[1]]
        s1 = s1 + jnp.sum(xh1, axis=1, keepdims=True)
        s2 = s2 + jnp.sum(xh1 * xh1, axis=1, keepdims=True)
        comm_ref[my] = jnp.concatenate([s1, s2], axis=1).T

        sends = []
        for d, peer in [(1, peers[1]), (0, peers[0]), (2, peers[2])]:
            rdma = pltpu.make_async_remote_copy(
                src_ref=comm_ref.at[my],
                dst_ref=comm_ref.at[my],
                send_sem=send_sems.at[d],
                recv_sem=recv_sems.at[my],
                device_id=(peer,),
                device_id_type=pl.DeviceIdType.MESH,
            )
            rdma.start()
            sends.append(rdma)

        g_load.wait()
        b_load.wait()
        g = g_vmem[:].reshape(1, -1)
        xf = x_vmem[:, :]
        xg = xf * g

        for d, peer in enumerate(peers):
            recv = pltpu.make_async_remote_copy(
                src_ref=comm_ref.at[peer],
                dst_ref=comm_ref.at[peer],
                send_sem=send_sems.at[d],
                recv_sem=recv_sems.at[peer],
                device_id=(peer,),
                device_id_type=pl.DeviceIdType.MESH,
            )
            recv.wait_recv()

        total = (
            comm_ref[0] + comm_ref[1] + comm_ref[2] + comm_ref[3]
        )
        mean = total[0:1, :].T / n_global
        var = total[1:2, :].T / n_global - mean * mean
        inv = lax.rsqrt(var + EPS)
        b = b_vmem[:].reshape(1, -1)
        out_vmem[:, :] = (xg * inv - (mean * inv) * g + b).astype(jnp.bfloat16)
        out_store = pltpu.make_async_copy(out_vmem, out_hbm, store_sem)
        out_store.start()
        out_store.wait()
        for rdma in sends:
            rdma.wait_send()

    return pl.pallas_call(
        body,
        out_shape=jax.ShapeDtypeStruct((m, n_local), jnp.bfloat16),
        in_specs=[
            pl.BlockSpec(memory_space=pltpu.MemorySpace.HBM),
            pl.BlockSpec(memory_space=pltpu.MemorySpace.HBM),
            pl.BlockSpec(memory_space=pltpu.MemorySpace.HBM),
        ],
        out_specs=pl.BlockSpec(memory_space=pltpu.MemorySpace.HBM),
        scratch_shapes=[
            pltpu.VMEM((m, n_local), jnp.float32),
            pltpu.VMEM((n_local,), jnp.float32),
            pltpu.VMEM((n_local,), jnp.float32),
            pltpu.VMEM((m, n_local), jnp.bfloat16),
            pltpu.VMEM((N_DEV, 2, m), jnp.float32),
            pltpu.SemaphoreType.DMA((4,)),
            pltpu.SemaphoreType.DMA,
            pltpu.SemaphoreType.DMA((N_DEV - 1,)),
            pltpu.SemaphoreType.DMA((N_DEV,)),
        ],
        compiler_params=pltpu.CompilerParams(collective_id=0),
    )(
        pltpu.with_memory_space_constraint(x, pltpu.MemorySpace.HBM),
        pltpu.with_memory_space_constraint(gamma, pltpu.MemorySpace.HBM),
        pltpu.with_memory_space_constraint(beta, pltpu.MemorySpace.HBM),
    )


# device time: 8055 ns/iter; 1.4428x vs baseline; 1.0174x over previous
import jax
import jax.numpy as jnp
from jax import lax
from jax.experimental import pallas as pl
from jax.experimental.pallas import tpu as pltpu

N_DEV = 4
EPS = 1e-5


def kernel(x, gamma, beta):
    m, n_local = x.shape
    n_global = n_local * N_DEV
    mh = m // 2

    def body(
        x_hbm, g_hbm, b_hbm, out_hbm,
        x_vmem, g_vmem, b_vmem, out_vmem,
        comm_ref, load_sems, store_sem, send_sems, recv_sems,
    ):
        my = lax.axis_index("i")
        peers = [lax.rem(my + d, N_DEV) for d in range(1, N_DEV)]

        halves = [pl.ds(0, mh), pl.ds(mh, mh)]
        x_loads = []
        for c, rows in enumerate(halves):
            cp = pltpu.make_async_copy(
                x_hbm.at[rows], x_vmem.at[rows], load_sems.at[c]
            )
            cp.start()
            x_loads.append(cp)
        g_load = pltpu.make_async_copy(g_hbm, g_vmem, load_sems.at[2])
        b_load = pltpu.make_async_copy(b_hbm, b_vmem, load_sems.at[3])
        g_load.start()
        b_load.start()

        barrier_sem = pltpu.get_barrier_semaphore()
        for peer in peers:
            pl.semaphore_signal(
                barrier_sem, inc=1,
                device_id=(peer,), device_id_type=pl.DeviceIdType.MESH,
            )
        pl.semaphore_wait(barrier_sem, N_DEV - 1)

        x_loads[0].wait()
        xh0 = x_vmem[halves[0], :]
        st0 = jnp.concatenate(
            [
                jnp.sum(xh0, axis=1, keepdims=True),
                jnp.sum(xh0 * xh0, axis=1, keepdims=True),
            ],
            axis=1,
        ).T
        x_loads[1].wait()
        xh1 = x_vmem[halves[1], :]
        st1 = jnp.concatenate(
            [
                jnp.sum(xh1, axis=1, keepdims=True),
                jnp.sum(xh1 * xh1, axis=1, keepdims=True),
            ],
            axis=1,
        ).T
        comm_ref[my] = jnp.concatenate([st0, st1], axis=1)

        sends = []
        for d, peer in [(1, peers[1]), (0, peers[0]), (2, peers[2])]:
            rdma = pltpu.make_async_remote_copy(
                src_ref=comm_ref.at[my],
                dst_ref=comm_ref.at[my],
                send_sem=send_sems.at[d],
                recv_sem=recv_sems.at[my],
                device_id=(peer,),
                device_id_type=pl.DeviceIdType.MESH,
            )
            rdma.start()
            sends.append(rdma)

        g_load.wait()
        b_load.wait()
        g = g_vmem[:].reshape(1, -1)
        xf = x_vmem[:, :]
        xg = xf * g

        for d, peer in enumerate(peers):
            recv = pltpu.make_async_remote_copy(
                src_ref=comm_ref.at[peer],
                dst_ref=comm_ref.at[peer],
                send_sem=send_sems.at[d],
                recv_sem=recv_sems.at[peer],
                device_id=(peer,),
                device_id_type=pl.DeviceIdType.MESH,
            )
            recv.wait_recv()

        total = (
            comm_ref[0] + comm_ref[1] + comm_ref[2] + comm_ref[3]
        )
        mean = total[0:1, :].T / n_global
        var = total[1:2, :].T / n_global - mean * mean
        inv = lax.rsqrt(var + EPS)
        b = b_vmem[:].reshape(1, -1)
        out_vmem[:, :] = (xg * inv - (mean * inv) * g + b).astype(jnp.bfloat16)
        out_store = pltpu.make_async_copy(out_vmem, out_hbm, store_sem)
        out_store.start()
        out_store.wait()
        for rdma in sends:
            rdma.wait_send()

    return pl.pallas_call(
        body,
        out_shape=jax.ShapeDtypeStruct((m, n_local), jnp.bfloat16),
        in_specs=[
            pl.BlockSpec(memory_space=pltpu.MemorySpace.HBM),
            pl.BlockSpec(memory_space=pltpu.MemorySpace.HBM),
            pl.BlockSpec(memory_space=pltpu.MemorySpace.HBM),
        ],
        out_specs=pl.BlockSpec(memory_space=pltpu.MemorySpace.HBM),
        scratch_shapes=[
            pltpu.VMEM((m, n_local), jnp.float32),
            pltpu.VMEM((n_local,), jnp.float32),
            pltpu.VMEM((n_local,), jnp.float32),
            pltpu.VMEM((m, n_local), jnp.bfloat16),
            pltpu.VMEM((N_DEV, 2, m), jnp.float32),
            pltpu.SemaphoreType.DMA((4,)),
            pltpu.SemaphoreType.DMA,
            pltpu.SemaphoreType.DMA((N_DEV - 1,)),
            pltpu.SemaphoreType.DMA((N_DEV,)),
        ],
        compiler_params=pltpu.CompilerParams(collective_id=0),
    )(
        pltpu.with_memory_space_constraint(x, pltpu.MemorySpace.HBM),
        pltpu.with_memory_space_constraint(gamma, pltpu.MemorySpace.HBM),
        pltpu.with_memory_space_constraint(beta, pltpu.MemorySpace.HBM),
    )
